# baseline (speedup 1.0000x reference)
import functools

import jax
import jax.numpy as jnp
from jax import lax
from jax.experimental import pallas as pl
from jax.experimental.pallas import tpu as pltpu
from jax.experimental.pallas import tpu_sc as plsc

F = 4
N = 50000
D = 128
T = 100000

CONSISTENCY_WEIGHT = 1.0
TEMPORAL_WEIGHT = 0.5

NPAD = 50176
TW = 50432
HALF = 50000
RANGE = 3136
BS = 64
NBLK = RANGE // BS
SCAN = 12544
SROW = 98

_mesh = plsc.VectorSubcoreMesh(core_axis_name="c", subcore_axis_name="s")


@functools.partial(
    pl.kernel,
    mesh=_mesh,
    out_type=jax.ShapeDtypeStruct((32, 128), jnp.float32),
    scratch_types=[
        pltpu.VMEM_SHARED((4 * TW,), jnp.int32),
        pltpu.VMEM((SCAN // 7,), jnp.int32),
        pltpu.VMEM((SCAN // 7,), jnp.int32),
        pltpu.VMEM((4 * (RANGE + 16),), jnp.int32),
        pltpu.VMEM((BS,), jnp.int32),
        pltpu.VMEM((BS,), jnp.int32),
        pltpu.VMEM((BS,), jnp.int32),
        pltpu.VMEM((BS,), jnp.int32),
        pltpu.VMEM((BS,), jnp.int32),
        pltpu.VMEM((BS,), jnp.int32),
        pltpu.VMEM((BS,), jnp.int32),
        pltpu.VMEM((BS,), jnp.int32),
        pltpu.VMEM((BS,), jnp.int32),
        pltpu.VMEM((BS,), jnp.int32),
        pltpu.VMEM((BS,), jnp.int32),
        pltpu.VMEM((BS,), jnp.int32),
        pltpu.VMEM((4, BS, 128), jnp.float32),
        pltpu.VMEM((4, BS, 128), jnp.float32),
        pltpu.VMEM((4, BS, 128), jnp.float32),
        pltpu.VMEM((128,), jnp.float32),
        pltpu.SemaphoreType.DMA,
        pltpu.SemaphoreType.DMA,
        pltpu.SemaphoreType.DMA,
    ],
)
def _cfc_sc(feat, pidx, out, tab, idxbuf, valbuf, invbuf,
            gA0, gA1, gA2, gA3, gB0, gB1, gB2, gB3, gC0, gC1, gC2, gC3,
            rbufA, rbufB, rbufC, obuf, semA, semB, semC):
    c = lax.axis_index("c")
    s = lax.axis_index("s")
    io16 = lax.iota(jnp.int32, 16)

    def fill_body(r, _):
        invbuf[pl.ds(r * 16, 16)] = jnp.full((16,), -1, jnp.int32)
        return 0
    lax.fori_loop(0, 197, fill_body, 0)
    for f4 in range(4):
        pltpu.sync_copy(invbuf.at[pl.ds(0, 3152)],
                        tab.at[pl.ds(f4 * TW + s * 3152, 3152)])

    f = s // 4
    q = s % 4
    cbase = c * HALF
    HS = SCAN // 7

    plsc.subcore_barrier()

    for h in range(7):
        pltpu.sync_copy(pidx.at[pl.ds(f * NPAD + q * SCAN + h * HS, HS)],
                        idxbuf)

        def scan_body(r, _):
            for k in range(8):
                lanepos = h * HS + r * 128 + k * 16 + io16
                iv = idxbuf[pl.ds(r * 128 + k * 16, 16)]
                local = iv - cbase
                inr = (local >= 0) & (local < HALF)
                trash = f * TW + NPAD + (lanepos & 255)
                dest = jnp.where(inr, f * TW + local, trash)
                val = q * SCAN + lanepos
                idxbuf[pl.ds(r * 128 + k * 16, 16)] = dest
                valbuf[pl.ds(r * 128 + k * 16, 16)] = val
            return 0
        lax.fori_loop(0, HS // 128, scan_body, 0)
        pltpu.sync_copy(valbuf, tab.at[idxbuf])

    plsc.subcore_barrier()

    for f4 in range(4):
        pltpu.sync_copy(tab.at[pl.ds(f4 * TW + s * RANGE, RANGE)],
                        invbuf.at[pl.ds(f4 * (RANGE + 16), RANGE)])

    zero16 = jnp.zeros((16,), jnp.float32)
    f32z = jnp.float32(0.0)
    R16 = RANGE + 16
    gA = (gA0, gA1, gA2, gA3)
    gB = (gB0, gB1, gB2, gB3)
    gC = (gC0, gC1, gC2, gC3)

    def fire(b, gxs, rbuf, sem):
        for f4 in range(4):
            for k in range(BS // 16):
                iv = invbuf[pl.ds(f4 * R16 + b * BS + k * 16, 16)]
                pres = iv >= 0
                slocal = s * RANGE + b * BS + k * 16 + io16
                spread = slocal & 32767
                gxs[f4][pl.ds(k * 16, 16)] = (
                    jnp.where(pres, iv, spread) + f4 * N)
        for f4 in range(4):
            pltpu.async_copy(feat.at[gxs[f4]], rbuf.at[f4], sem)

    def drain(rbuf, sem):
        for f4 in range(4):
            pltpu.make_async_copy(feat.at[pl.ds(0, BS)], rbuf.at[f4],
                                  sem).wait()

    def compute(b, rbuf, carry):
        def pt_body(j, icarry):
            iT, iP0, iP1, iP2, i01, i12, i23 = icarry
            jj = b * BS + j
            m0 = jnp.where(invbuf[pl.ds(jj, 16)][0] >= 0, 1.0, 0.0)
            m1 = jnp.where(invbuf[pl.ds(R16 + jj, 16)][0] >= 0, 1.0, 0.0)
            m2 = jnp.where(invbuf[pl.ds(2 * R16 + jj, 16)][0] >= 0, 1.0, 0.0)
            m3 = jnp.where(invbuf[pl.ds(3 * R16 + jj, 16)][0] >= 0, 1.0, 0.0)
            cnt = m0 + m1 + m2 + m3
            rc = jnp.where(cnt > 2.5,
                           jnp.where(cnt > 3.5, 0.25, jnp.float32(1.0 / 3.0)),
                           jnp.where(cnt > 1.5, 0.5,
                                     jnp.where(cnt > 0.5, 1.0, 0.0)))
            mp01 = m0 * m1
            mp02 = m0 * m2
            mp03 = m0 * m3
            mp12 = m1 * m2
            mp13 = m1 * m3
            mp23 = m2 * m3
            w01 = rc * mp01
            w02 = rc * mp02
            w03 = rc * mp03
            w12 = rc * mp12
            w13 = rc * mp13
            w23 = rc * mp23
            pd01 = zero16
            pd02 = zero16
            pd03 = zero16
            pd12 = zero16
            pd13 = zero16
            pd23 = zero16
            for k in range(8):
                r0 = rbuf[0, j, pl.ds(k * 16, 16)]
                r1 = rbuf[1, j, pl.ds(k * 16, 16)]
                r2 = rbuf[2, j, pl.ds(k * 16, 16)]
                r3 = rbuf[3, j, pl.ds(k * 16, 16)]
                d01 = r0 - r1
                d02 = r0 - r2
                d03 = r0 - r3
                d12 = r1 - r2
                d13 = r1 - r3
                d23 = r2 - r3
                pd01 = pd01 + d01 * d01
                pd02 = pd02 + d02 * d02
                pd03 = pd03 + d03 * d03
                pd12 = pd12 + d12 * d12
                pd13 = pd13 + d13 * d13
                pd23 = pd23 + d23 * d23
            iT = iT + (w01 * pd01 + w02 * pd02 + w03 * pd03)
            iT = iT + (w12 * pd12 + w13 * pd13 + w23 * pd23)
            return (iT,
                    iP0 + pd01 * mp01, iP1 + pd12 * mp12, iP2 + pd23 * mp23,
                    i01 + mp01, i12 + mp12, i23 + mp23)
        return lax.fori_loop(0, BS, pt_body, carry, unroll=1)

    bufs = ((gA, rbufA, semA), (gB, rbufB, semB), (gC, rbufC, semC))
    fire(0, *bufs[0])
    fire(1, *bufs[1])

    def tri_body(i, carry):
        for t in range(3):
            b = 3 * i + t
            gx, rb, sm = bufs[t]
            gx2, rb2, sm2 = bufs[(t + 2) % 3]
            drain(rb, sm)
            carry = compute(b, rb, carry)

            @pl.when(b + 2 <= NBLK - 1)
            def _():
                fire(b + 2, gx2, rb2, sm2)
        return carry

    init = (zero16, zero16, zero16, zero16, f32z, f32z, f32z)
    carry = lax.fori_loop(0, (NBLK - 1) // 3, tri_body, init)
    drain(rbufA, semA)
    accT, aP0, aP1, aP2, c01, c12, c23 = compute(NBLK - 1, rbufA, carry)

    cv = jnp.where(io16 == 0, c01, 0.0)
    cv = jnp.where(io16 == 1, c12, cv)
    cv = jnp.where(io16 == 2, c23, cv)
    obuf[pl.ds(0, 16)] = accT
    obuf[pl.ds(16, 16)] = jnp.zeros((16,), jnp.float32)
    obuf[pl.ds(32, 16)] = aP0
    obuf[pl.ds(48, 16)] = aP1
    obuf[pl.ds(64, 16)] = aP2
    obuf[pl.ds(80, 16)] = cv
    obuf[pl.ds(96, 16)] = jnp.zeros((16,), jnp.float32)
    obuf[pl.ds(112, 16)] = jnp.zeros((16,), jnp.float32)
    wid = c * 16 + s
    pltpu.sync_copy(obuf, out.at[wid])


def kernel(point_features, visibility_matrix, point_indices):
    del visibility_matrix
    feat = point_features.reshape(F * N, D)
    pidx = jnp.pad(point_indices.astype(jnp.int32), ((0, 0), (0, NPAD - N)),
                   constant_values=-1).reshape(F * NPAD)
    parts = _cfc_sc(feat, pidx)
    Tnum = jnp.sum(parts[:, 0:16])
    P = jnp.stack([jnp.sum(parts[:, 32:48]), jnp.sum(parts[:, 48:64]),
                   jnp.sum(parts[:, 64:80])])
    cnt = jnp.sum(parts[:, 80:83], axis=0)
    temporal = Tnum / jnp.float32(F * N * D)
    mse = jnp.where(cnt > 0, P / (jnp.maximum(cnt, 1.0) * D), 0.0)
    consistency = jnp.mean(mse)
    total = CONSISTENCY_WEIGHT * consistency + TEMPORAL_WEIGHT * temporal
    return (consistency, temporal, total)


# speedup vs baseline: 1.0042x; 1.0042x over previous
import functools

import jax
import jax.numpy as jnp
from jax import lax
from jax.experimental import pallas as pl
from jax.experimental.pallas import tpu as pltpu
from jax.experimental.pallas import tpu_sc as plsc

F = 4
N = 50000
D = 128
T = 100000

CONSISTENCY_WEIGHT = 1.0
TEMPORAL_WEIGHT = 0.5

NPAD = 50176
TW = 50432
HALF = 50000
RANGE = 3136
BS = 64
NBLK = RANGE // BS
SCAN = 12544
SROW = 98

_mesh = plsc.VectorSubcoreMesh(core_axis_name="c", subcore_axis_name="s")


@functools.partial(
    pl.kernel,
    mesh=_mesh,
    out_type=jax.ShapeDtypeStruct((32, 128), jnp.float32),
    scratch_types=[
        pltpu.VMEM_SHARED((4 * TW,), jnp.int32),
        pltpu.VMEM((SCAN // 7,), jnp.int32),
        pltpu.VMEM((SCAN // 7,), jnp.int32),
        pltpu.VMEM((4 * (RANGE + 16),), jnp.int32),
        pltpu.VMEM((BS,), jnp.int32),
        pltpu.VMEM((BS,), jnp.int32),
        pltpu.VMEM((BS,), jnp.int32),
        pltpu.VMEM((BS,), jnp.int32),
        pltpu.VMEM((BS,), jnp.int32),
        pltpu.VMEM((BS,), jnp.int32),
        pltpu.VMEM((BS,), jnp.int32),
        pltpu.VMEM((BS,), jnp.int32),
        pltpu.VMEM((4, BS, 128), jnp.float32),
        pltpu.VMEM((4, BS, 128), jnp.float32),
        pltpu.VMEM((128,), jnp.float32),
        pltpu.SemaphoreType.DMA,
        pltpu.SemaphoreType.DMA,
    ],
)
def _cfc_sc(feat, pidx, out, tab, idxbuf, valbuf, invbuf,
            gA0, gA1, gA2, gA3, gB0, gB1, gB2, gB3,
            rbufA, rbufB, obuf, semA, semB):
    c = lax.axis_index("c")
    s = lax.axis_index("s")
    io16 = lax.iota(jnp.int32, 16)

    def fill_body(r, _):
        invbuf[pl.ds(r * 16, 16)] = jnp.full((16,), -1, jnp.int32)
        return 0
    lax.fori_loop(0, 197, fill_body, 0)
    for f4 in range(4):
        pltpu.sync_copy(invbuf.at[pl.ds(0, 3152)],
                        tab.at[pl.ds(f4 * TW + s * 3152, 3152)])

    f = s // 4
    q = s % 4
    cbase = c * HALF
    HS = SCAN // 7

    plsc.subcore_barrier()

    for h in range(7):
        pltpu.sync_copy(pidx.at[pl.ds(f * NPAD + q * SCAN + h * HS, HS)],
                        idxbuf)

        def scan_body(r, _):
            for k in range(8):
                lanepos = h * HS + r * 128 + k * 16 + io16
                iv = idxbuf[pl.ds(r * 128 + k * 16, 16)]
                local = iv - cbase
                inr = (local >= 0) & (local < HALF)
                trash = f * TW + NPAD + (lanepos & 255)
                dest = jnp.where(inr, f * TW + local, trash)
                val = q * SCAN + lanepos
                idxbuf[pl.ds(r * 128 + k * 16, 16)] = dest
                valbuf[pl.ds(r * 128 + k * 16, 16)] = val
            return 0
        lax.fori_loop(0, HS // 128, scan_body, 0)
        pltpu.sync_copy(valbuf, tab.at[idxbuf])

    plsc.subcore_barrier()

    for f4 in range(4):
        pltpu.sync_copy(tab.at[pl.ds(f4 * TW + s * RANGE, RANGE)],
                        invbuf.at[pl.ds(f4 * (RANGE + 16), RANGE)])

    zero16 = jnp.zeros((16,), jnp.float32)
    f32z = jnp.float32(0.0)
    R16 = RANGE + 16
    gA = (gA0, gA1, gA2, gA3)
    gB = (gB0, gB1, gB2, gB3)

    def fire(b, gxs, rbuf, sem):
        for f4 in range(4):
            for k in range(BS // 16):
                iv = invbuf[pl.ds(f4 * R16 + b * BS + k * 16, 16)]
                pres = iv >= 0
                slocal = s * RANGE + b * BS + k * 16 + io16
                spread = slocal & 32767
                gxs[f4][pl.ds(k * 16, 16)] = (
                    jnp.where(pres, iv, spread) + f4 * N)
        for f4 in range(4):
            pltpu.async_copy(feat.at[gxs[f4]], rbuf.at[f4], sem)

    def drain(rbuf, sem):
        for f4 in range(4):
            pltpu.make_async_copy(feat.at[pl.ds(0, BS)], rbuf.at[f4],
                                  sem).wait()

    def compute(b, rbuf, carry):
        def pt_body(j, icarry):
            iT, iP0, iP1, iP2, i01, i12, i23 = icarry
            jj = b * BS + j
            m0 = jnp.where(invbuf[pl.ds(jj, 16)][0] >= 0, 1.0, 0.0)
            m1 = jnp.where(invbuf[pl.ds(R16 + jj, 16)][0] >= 0, 1.0, 0.0)
            m2 = jnp.where(invbuf[pl.ds(2 * R16 + jj, 16)][0] >= 0, 1.0, 0.0)
            m3 = jnp.where(invbuf[pl.ds(3 * R16 + jj, 16)][0] >= 0, 1.0, 0.0)
            cnt = m0 + m1 + m2 + m3
            rc = jnp.where(cnt > 2.5,
                           jnp.where(cnt > 3.5, 0.25, jnp.float32(1.0 / 3.0)),
                           jnp.where(cnt > 1.5, 0.5,
                                     jnp.where(cnt > 0.5, 1.0, 0.0)))
            mp01 = m0 * m1
            mp02 = m0 * m2
            mp03 = m0 * m3
            mp12 = m1 * m2
            mp13 = m1 * m3
            mp23 = m2 * m3
            w01 = rc * mp01
            w02 = rc * mp02
            w03 = rc * mp03
            w12 = rc * mp12
            w13 = rc * mp13
            w23 = rc * mp23
            pd01 = zero16
            pd02 = zero16
            pd03 = zero16
            pd12 = zero16
            pd13 = zero16
            pd23 = zero16
            for k in range(8):
                r0 = rbuf[0, j, pl.ds(k * 16, 16)]
                r1 = rbuf[1, j, pl.ds(k * 16, 16)]
                r2 = rbuf[2, j, pl.ds(k * 16, 16)]
                r3 = rbuf[3, j, pl.ds(k * 16, 16)]
                d01 = r0 - r1
                d02 = r0 - r2
                d03 = r0 - r3
                d12 = r1 - r2
                d13 = r1 - r3
                d23 = r2 - r3
                pd01 = pd01 + d01 * d01
                pd02 = pd02 + d02 * d02
                pd03 = pd03 + d03 * d03
                pd12 = pd12 + d12 * d12
                pd13 = pd13 + d13 * d13
                pd23 = pd23 + d23 * d23
            iT = iT + (w01 * pd01 + w02 * pd02 + w03 * pd03)
            iT = iT + (w12 * pd12 + w13 * pd13 + w23 * pd23)
            return (iT,
                    iP0 + pd01 * mp01, iP1 + pd12 * mp12, iP2 + pd23 * mp23,
                    i01 + mp01, i12 + mp12, i23 + mp23)
        return lax.fori_loop(0, BS, pt_body, carry, unroll=1)

    fire(0, gA, rbufA, semA)

    def dbl_body(i, carry):
        b0 = 2 * i
        fire(b0 + 1, gB, rbufB, semB)
        drain(rbufA, semA)
        carry = compute(b0, rbufA, carry)
        fire(b0 + 2, gA, rbufA, semA)
        drain(rbufB, semB)
        carry = compute(b0 + 1, rbufB, carry)
        return carry

    init = (zero16, zero16, zero16, zero16, f32z, f32z, f32z)
    carry = lax.fori_loop(0, (NBLK - 1) // 2, dbl_body, init)
    drain(rbufA, semA)
    accT, aP0, aP1, aP2, c01, c12, c23 = compute(NBLK - 1, rbufA, carry)

    cv = jnp.where(io16 == 0, c01, 0.0)
    cv = jnp.where(io16 == 1, c12, cv)
    cv = jnp.where(io16 == 2, c23, cv)
    obuf[pl.ds(0, 16)] = accT
    obuf[pl.ds(16, 16)] = jnp.zeros((16,), jnp.float32)
    obuf[pl.ds(32, 16)] = aP0
    obuf[pl.ds(48, 16)] = aP1
    obuf[pl.ds(64, 16)] = aP2
    obuf[pl.ds(80, 16)] = cv
    obuf[pl.ds(96, 16)] = jnp.zeros((16,), jnp.float32)
    obuf[pl.ds(112, 16)] = jnp.zeros((16,), jnp.float32)
    wid = c * 16 + s
    pltpu.sync_copy(obuf, out.at[wid])


def kernel(point_features, visibility_matrix, point_indices):
    del visibility_matrix
    feat = point_features.reshape(F * N, D)
    pidx = jnp.pad(point_indices.astype(jnp.int32), ((0, 0), (0, NPAD - N)),
                   constant_values=-1).reshape(F * NPAD)
    parts = _cfc_sc(feat, pidx)
    Tnum = jnp.sum(parts[:, 0:16])
    P = jnp.stack([jnp.sum(parts[:, 32:48]), jnp.sum(parts[:, 48:64]),
                   jnp.sum(parts[:, 64:80])])
    cnt = jnp.sum(parts[:, 80:83], axis=0)
    temporal = Tnum / jnp.float32(F * N * D)
    mse = jnp.where(cnt > 0, P / (jnp.maximum(cnt, 1.0) * D), 0.0)
    consistency = jnp.mean(mse)
    total = CONSISTENCY_WEIGHT * consistency + TEMPORAL_WEIGHT * temporal
    return (consistency, temporal, total)


# speedup vs baseline: 1.0394x; 1.0351x over previous
import functools

import jax
import jax.numpy as jnp
from jax import lax
from jax.experimental import pallas as pl
from jax.experimental.pallas import tpu as pltpu
from jax.experimental.pallas import tpu_sc as plsc

F = 4
N = 50000
D = 128
T = 100000

CONSISTENCY_WEIGHT = 1.0
TEMPORAL_WEIGHT = 0.5

NPAD = 50176
TW = 51200
HALF = 50000
RANGE = 3136
BS = 64
NBLK = RANGE // BS
SCAN = 12544
SROW = 98

_mesh = plsc.VectorSubcoreMesh(core_axis_name="c", subcore_axis_name="s")


@functools.partial(
    pl.kernel,
    mesh=_mesh,
    out_type=jax.ShapeDtypeStruct((32, 128), jnp.float32),
    scratch_types=[
        pltpu.VMEM_SHARED((4 * TW,), jnp.int32),
        pltpu.VMEM((SCAN // 2,), jnp.int32),
        pltpu.VMEM((SCAN // 2,), jnp.int32),
        pltpu.VMEM((4 * (RANGE + 16),), jnp.int32),
        pltpu.VMEM((BS,), jnp.int32),
        pltpu.VMEM((BS,), jnp.int32),
        pltpu.VMEM((BS,), jnp.int32),
        pltpu.VMEM((BS,), jnp.int32),
        pltpu.VMEM((BS,), jnp.int32),
        pltpu.VMEM((BS,), jnp.int32),
        pltpu.VMEM((BS,), jnp.int32),
        pltpu.VMEM((BS,), jnp.int32),
        pltpu.VMEM((4, BS, 128), jnp.float32),
        pltpu.VMEM((4, BS, 128), jnp.float32),
        pltpu.VMEM((128,), jnp.float32),
        pltpu.SemaphoreType.DMA,
        pltpu.SemaphoreType.DMA,
    ],
)
def _cfc_sc(feat, pidx, out, tab, idxbuf, valbuf, invbuf,
            gA0, gA1, gA2, gA3, gB0, gB1, gB2, gB3,
            rbufA, rbufB, obuf, semA, semB):
    c = lax.axis_index("c")
    s = lax.axis_index("s")
    io16 = lax.iota(jnp.int32, 16)

    def fill_body(r, _):
        invbuf[pl.ds(r * 16, 16)] = jnp.full((16,), -1, jnp.int32)
        return 0
    lax.fori_loop(0, 200, fill_body, 0)
    for f4 in range(4):
        pltpu.sync_copy(invbuf.at[pl.ds(0, 3200)],
                        tab.at[pl.ds(f4 * TW + s * 3200, 3200)])

    f = s // 4
    q = s % 4
    cbase = c * HALF
    HS = SCAN // 2

    plsc.subcore_barrier()

    for h in range(2):
        pltpu.sync_copy(pidx.at[pl.ds(f * NPAD + q * SCAN + h * HS, HS)],
                        idxbuf)

        def scan_body(r, _):
            for k in range(8):
                lanepos = h * HS + r * 128 + k * 16 + io16
                iv = idxbuf[pl.ds(r * 128 + k * 16, 16)]
                local = iv - cbase
                inr = (local >= 0) & (local < HALF)
                trash = f * TW + NPAD + (lanepos & 1023)
                dest = jnp.where(inr, f * TW + local, trash)
                val = q * SCAN + lanepos
                idxbuf[pl.ds(r * 128 + k * 16, 16)] = dest
                valbuf[pl.ds(r * 128 + k * 16, 16)] = val
            return 0
        lax.fori_loop(0, HS // 128, scan_body, 0)
        pltpu.sync_copy(valbuf, tab.at[idxbuf])

    plsc.subcore_barrier()

    for f4 in range(4):
        pltpu.sync_copy(tab.at[pl.ds(f4 * TW + s * RANGE, RANGE)],
                        invbuf.at[pl.ds(f4 * (RANGE + 16), RANGE)])

    zero16 = jnp.zeros((16,), jnp.float32)
    f32z = jnp.float32(0.0)
    R16 = RANGE + 16
    gA = (gA0, gA1, gA2, gA3)
    gB = (gB0, gB1, gB2, gB3)

    def fire(b, gxs, rbuf, sem):
        for f4 in range(4):
            for k in range(BS // 16):
                iv = invbuf[pl.ds(f4 * R16 + b * BS + k * 16, 16)]
                pres = iv >= 0
                slocal = s * RANGE + b * BS + k * 16 + io16
                spread = slocal & 32767
                gxs[f4][pl.ds(k * 16, 16)] = (
                    jnp.where(pres, iv, spread) + f4 * N)
        for f4 in range(4):
            pltpu.async_copy(feat.at[gxs[f4]], rbuf.at[f4], sem)

    def drain(rbuf, sem):
        for f4 in range(4):
            pltpu.make_async_copy(feat.at[pl.ds(0, BS)], rbuf.at[f4],
                                  sem).wait()

    def compute(b, rbuf, carry):
        def pt_body(j, icarry):
            iT, iP0, iP1, iP2, i01, i12, i23 = icarry
            jj = b * BS + j
            m0 = jnp.where(invbuf[pl.ds(jj, 16)][0] >= 0, 1.0, 0.0)
            m1 = jnp.where(invbuf[pl.ds(R16 + jj, 16)][0] >= 0, 1.0, 0.0)
            m2 = jnp.where(invbuf[pl.ds(2 * R16 + jj, 16)][0] >= 0, 1.0, 0.0)
            m3 = jnp.where(invbuf[pl.ds(3 * R16 + jj, 16)][0] >= 0, 1.0, 0.0)
            cnt = m0 + m1 + m2 + m3
            rc = jnp.where(cnt > 2.5,
                           jnp.where(cnt > 3.5, 0.25, jnp.float32(1.0 / 3.0)),
                           jnp.where(cnt > 1.5, 0.5,
                                     jnp.where(cnt > 0.5, 1.0, 0.0)))
            mp01 = m0 * m1
            mp02 = m0 * m2
            mp03 = m0 * m3
            mp12 = m1 * m2
            mp13 = m1 * m3
            mp23 = m2 * m3
            w01 = rc * mp01
            w02 = rc * mp02
            w03 = rc * mp03
            w12 = rc * mp12
            w13 = rc * mp13
            w23 = rc * mp23
            pd01 = zero16
            pd02 = zero16
            pd03 = zero16
            pd12 = zero16
            pd13 = zero16
            pd23 = zero16
            for k in range(8):
                r0 = rbuf[0, j, pl.ds(k * 16, 16)]
                r1 = rbuf[1, j, pl.ds(k * 16, 16)]
                r2 = rbuf[2, j, pl.ds(k * 16, 16)]
                r3 = rbuf[3, j, pl.ds(k * 16, 16)]
                d01 = r0 - r1
                d02 = r0 - r2
                d03 = r0 - r3
                d12 = r1 - r2
                d13 = r1 - r3
                d23 = r2 - r3
                pd01 = pd01 + d01 * d01
                pd02 = pd02 + d02 * d02
                pd03 = pd03 + d03 * d03
                pd12 = pd12 + d12 * d12
                pd13 = pd13 + d13 * d13
                pd23 = pd23 + d23 * d23
            iT = iT + (w01 * pd01 + w02 * pd02 + w03 * pd03)
            iT = iT + (w12 * pd12 + w13 * pd13 + w23 * pd23)
            return (iT,
                    iP0 + pd01 * mp01, iP1 + pd12 * mp12, iP2 + pd23 * mp23,
                    i01 + mp01, i12 + mp12, i23 + mp23)
        return lax.fori_loop(0, BS, pt_body, carry, unroll=1)

    fire(0, gA, rbufA, semA)

    def dbl_body(i, carry):
        b0 = 2 * i
        fire(b0 + 1, gB, rbufB, semB)
        drain(rbufA, semA)
        carry = compute(b0, rbufA, carry)
        fire(b0 + 2, gA, rbufA, semA)
        drain(rbufB, semB)
        carry = compute(b0 + 1, rbufB, carry)
        return carry

    init = (zero16, zero16, zero16, zero16, f32z, f32z, f32z)
    carry = lax.fori_loop(0, (NBLK - 1) // 2, dbl_body, init)
    drain(rbufA, semA)
    accT, aP0, aP1, aP2, c01, c12, c23 = compute(NBLK - 1, rbufA, carry)

    cv = jnp.where(io16 == 0, c01, 0.0)
    cv = jnp.where(io16 == 1, c12, cv)
    cv = jnp.where(io16 == 2, c23, cv)
    obuf[pl.ds(0, 16)] = accT
    obuf[pl.ds(16, 16)] = jnp.zeros((16,), jnp.float32)
    obuf[pl.ds(32, 16)] = aP0
    obuf[pl.ds(48, 16)] = aP1
    obuf[pl.ds(64, 16)] = aP2
    obuf[pl.ds(80, 16)] = cv
    obuf[pl.ds(96, 16)] = jnp.zeros((16,), jnp.float32)
    obuf[pl.ds(112, 16)] = jnp.zeros((16,), jnp.float32)
    wid = c * 16 + s
    pltpu.sync_copy(obuf, out.at[wid])


def kernel(point_features, visibility_matrix, point_indices):
    del visibility_matrix
    feat = point_features.reshape(F * N, D)
    pidx = jnp.pad(point_indices.astype(jnp.int32), ((0, 0), (0, NPAD - N)),
                   constant_values=-1).reshape(F * NPAD)
    parts = _cfc_sc(feat, pidx)
    Tnum = jnp.sum(parts[:, 0:16])
    P = jnp.stack([jnp.sum(parts[:, 32:48]), jnp.sum(parts[:, 48:64]),
                   jnp.sum(parts[:, 64:80])])
    cnt = jnp.sum(parts[:, 80:83], axis=0)
    temporal = Tnum / jnp.float32(F * N * D)
    mse = jnp.where(cnt > 0, P / (jnp.maximum(cnt, 1.0) * D), 0.0)
    consistency = jnp.mean(mse)
    total = CONSISTENCY_WEIGHT * consistency + TEMPORAL_WEIGHT * temporal
    return (consistency, temporal, total)
